# Initial kernel scaffold; baseline (speedup 1.0000x reference)
import jax
import jax.numpy as jnp
from jax.experimental import pallas as pl


def kernel(input, weight):
    raise NotImplementedError("write your pallas kernel here")


# speedup vs baseline: 1.0942x; 1.0942x over previous
import functools

import jax
import jax.numpy as jnp
from jax import lax
from jax.experimental import pallas as pl
from jax.experimental.pallas import tpu as pltpu
from jax.experimental.pallas import tpu_sc as plsc

EMBEDDING_DIM = 32
CHUNK = 1024


@functools.lru_cache(maxsize=None)
def _make_gather(batch: int, dim: int):
    info = plsc.get_sparse_core_info()
    nw = info.num_cores * info.num_subcores
    assert batch % nw == 0
    b_per_w = batch // nw
    chunk = min(CHUNK, b_per_w)
    assert b_per_w % chunk == 0
    n_chunks = b_per_w // chunk
    mesh = plsc.VectorSubcoreMesh(core_axis_name="c", subcore_axis_name="s")

    @functools.partial(
        pl.kernel,
        mesh=mesh,
        out_type=jax.ShapeDtypeStruct((batch, dim), jnp.float32),
        scratch_types=[
            pltpu.VMEM((chunk,), jnp.int32),
            pltpu.VMEM((chunk, dim), jnp.float32),
            pltpu.SemaphoreType.DMA,
        ],
        compiler_params=pltpu.CompilerParams(use_tc_tiling_on_sc=False),
    )
    def gather_kernel(idx_hbm, table_hbm, out_hbm, idx_v, rows_v, sem):
        wid = lax.axis_index("s") * info.num_cores + lax.axis_index("c")
        base = wid * b_per_w

        def body(i, carry):
            off = base + i * chunk
            pltpu.sync_copy(idx_hbm.at[pl.ds(off, chunk)], idx_v)
            pltpu.async_copy(table_hbm.at[idx_v], rows_v, sem).wait()
            pltpu.sync_copy(rows_v, out_hbm.at[pl.ds(off, chunk)])
            return carry

        lax.fori_loop(0, n_chunks, body, 0)

    return gather_kernel


@jax.jit
def kernel(input, weight):
    b, h = input.shape
    dim = weight.shape[1]
    idx = input.reshape(b * h).astype(jnp.int32)
    out = _make_gather(b * h, dim)(idx, weight)
    return out.reshape(b, h, dim)


# speedup vs baseline: 1.1134x; 1.0175x over previous
import functools

import jax
import jax.numpy as jnp
from jax import lax
from jax.experimental import pallas as pl
from jax.experimental.pallas import tpu as pltpu
from jax.experimental.pallas import tpu_sc as plsc

CHUNK = 800
NBUF = 4


@functools.lru_cache(maxsize=None)
def _make_gather(batch: int, dim: int):
    info = plsc.get_sparse_core_info()
    nw = info.num_cores * info.num_subcores
    assert batch % nw == 0
    b_per_w = batch // nw
    chunk = min(CHUNK, b_per_w)
    assert b_per_w % (chunk * NBUF) == 0
    n_chunks = b_per_w // chunk
    n_outer = n_chunks // NBUF
    mesh = plsc.VectorSubcoreMesh(core_axis_name="c", subcore_axis_name="s")

    scratch = (
        [pltpu.VMEM((chunk,), jnp.int32)] * NBUF
        + [pltpu.VMEM((chunk, dim), jnp.float32)] * NBUF
        + [pltpu.SemaphoreType.DMA] * (3 * NBUF)
    )

    @functools.partial(
        pl.kernel,
        mesh=mesh,
        out_type=jax.ShapeDtypeStruct((batch, dim), jnp.float32),
        scratch_types=scratch,
        compiler_params=pltpu.CompilerParams(use_tc_tiling_on_sc=False),
    )
    def gather_kernel(idx_hbm, table_hbm, out_hbm, *s):
        idxb = s[0:NBUF]
        rowb = s[NBUF:2 * NBUF]
        semi = s[2 * NBUF:3 * NBUF]
        semg = s[3 * NBUF:4 * NBUF]
        semo = s[4 * NBUF:5 * NBUF]

        wid = lax.axis_index("s") * info.num_cores + lax.axis_index("c")
        base = wid * b_per_w

        def start_idx(c, b):
            pltpu.async_copy(idx_hbm.at[pl.ds(base + c * chunk, chunk)],
                             idxb[b], semi[b])

        def wait_idx(b):
            pltpu.make_async_copy(idx_hbm.at[pl.ds(base, chunk)],
                                  idxb[b], semi[b]).wait()

        def start_gather(b):
            pltpu.async_copy(table_hbm.at[idxb[b]], rowb[b], semg[b])

        def wait_gather(b):
            pltpu.make_async_copy(table_hbm.at[idxb[b]],
                                  rowb[b], semg[b]).wait()

        def start_out(c, b):
            pltpu.async_copy(rowb[b],
                             out_hbm.at[pl.ds(base + c * chunk, chunk)],
                             semo[b])

        def wait_out(b):
            pltpu.make_async_copy(rowb[b],
                                  out_hbm.at[pl.ds(base, chunk)],
                                  semo[b]).wait()

        for b in range(NBUF):
            start_idx(b, b)

        @pl.loop(0, n_outer)
        def _outer(g):
            for b in range(NBUF):
                c = g * NBUF + b
                bp = (b + NBUF - 1) % NBUF

                wait_idx(b)

                @pl.when(g >= 1)
                def _():
                    wait_out(b)

                start_gather(b)

                if b >= 1:
                    wait_gather(bp)
                    start_out(c - 1, bp)

                    @pl.when(c + NBUF - 1 < n_chunks)
                    def _():
                        start_idx(c + NBUF - 1, bp)
                else:
                    @pl.when(g >= 1)
                    def _():
                        wait_gather(bp)
                        start_out(c - 1, bp)

                    @pl.when((g >= 1) & (c + NBUF - 1 < n_chunks))
                    def _():
                        start_idx(c + NBUF - 1, bp)

        last_b = (n_chunks - 1) % NBUF
        wait_gather(last_b)
        start_out(n_chunks - 1, last_b)
        for b in range(NBUF):
            wait_out(b)

    return gather_kernel


@jax.jit
def kernel(input, weight):
    b, h = input.shape
    dim = weight.shape[1]
    idx = input.reshape(b * h).astype(jnp.int32)
    out = _make_gather(b * h, dim)(idx, weight)
    return out.reshape(b, h, dim)


# speedup vs baseline: 1.2139x; 1.0903x over previous
import functools

import jax
import jax.numpy as jnp
from jax import lax
from jax.experimental import pallas as pl
from jax.experimental.pallas import tpu as pltpu
from jax.experimental.pallas import tpu_sc as plsc

LANES = 16


def _iota16():
    return lax.iota(jnp.int32, LANES)


@functools.lru_cache(maxsize=None)
def _make_pack(v: int, d: int):
    info = plsc.get_sparse_core_info()
    nc, ns = info.num_cores, info.num_subcores
    nw = nc * ns
    assert d == 32 and v % 32 == 0
    nj = v // 128
    tail = v % 128
    per = nj // nw
    rem = nj % nw
    mesh = plsc.VectorSubcoreMesh(core_axis_name="c", subcore_axis_name="s")

    @functools.partial(
        pl.kernel,
        mesh=mesh,
        out_type=jax.ShapeDtypeStruct((v // 4, 128), jnp.float32),
        scratch_types=[
            pltpu.VMEM((d, 128), jnp.float32),
            pltpu.VMEM((d, 128), jnp.float32),
            pltpu.VMEM((32, 128), jnp.float32),
            pltpu.VMEM((32, 128), jnp.float32),
            pltpu.SemaphoreType.DMA,
            pltpu.SemaphoreType.DMA,
            pltpu.SemaphoreType.DMA,
            pltpu.SemaphoreType.DMA,
        ],
        compiler_params=pltpu.CompilerParams(needs_layout_passes=False),
    )
    def pack_kernel(w2_hbm, t4_hbm, in0, in1, out0, out1, si0, si1, so0, so1):
        wid = lax.axis_index("s") * nc + lax.axis_index("c")
        inb, outb, si, so = [in0, in1], [out0, out1], [si0, si1], [so0, so1]
        nblk = per + jnp.where(wid < rem, 1, 0)

        def start_in(m, b):
            j = m * nw + wid
            pltpu.async_copy(w2_hbm.at[:, pl.ds(j * 128, 128)], inb[b], si[b])

        def wait_in(b):
            pltpu.make_async_copy(w2_hbm.at[:, pl.ds(0, 128)],
                                  inb[b], si[b]).wait()

        def start_out(m, b):
            j = m * nw + wid
            pltpu.async_copy(outb[b], t4_hbm.at[pl.ds(j * 32, 32)], so[b])

        def wait_out(b):
            pltpu.make_async_copy(outb[b], t4_hbm.at[pl.ds(0, 32)],
                                  so[b]).wait()

        def transform(src, dst, nrows):
            for e in range(nrows):
                for s in range(4):
                    for h in range(2):
                        rows = _iota16() + LANES * h
                        cols = jnp.full((LANES,), 4 * e + s, jnp.int32)
                        dst[e, pl.ds(32 * s + LANES * h, LANES)] = (
                            plsc.load_gather(src, [rows, cols]))

        start_in(0, 0)

        @pl.loop(0, per + 1)
        def _blocks(m):
            b = lax.rem(m, 2)

            @pl.when(m < nblk)
            def _():
                for bb in range(2):
                    @pl.when(b == bb)
                    def _():
                        @pl.when(m + 1 < nblk)
                        def _():
                            start_in(m + 1, 1 - bb)

                        wait_in(bb)

                        @pl.when(m >= 2)
                        def _():
                            wait_out(bb)

                        transform(inb[bb], outb[bb], 32)
                        start_out(m, bb)

        wait_out(0)
        wait_out(1)

        if tail:
            @pl.when(wid == rem)
            def _():
                for c in range(d):
                    pltpu.sync_copy(w2_hbm.at[c, pl.ds(nj * 128, tail)],
                                    in0.at[c, pl.ds(0, tail)])
                transform(in0, out0, tail // 4)
                pltpu.sync_copy(out0.at[pl.ds(0, tail // 4)],
                                t4_hbm.at[pl.ds(nj * 32, tail // 4)])

    return pack_kernel


@functools.lru_cache(maxsize=None)
def _make_lookup(t_len: int, b_len: int, d: int):
    info = plsc.get_sparse_core_info()
    nc, ns = info.num_cores, info.num_subcores
    nw = nc * ns
    nb = b_len // 128
    nblk = t_len * nb
    assert b_len % 128 == 0 and nblk % nw == 0
    per = nblk // nw
    mesh = plsc.VectorSubcoreMesh(core_axis_name="c", subcore_axis_name="s")

    @functools.partial(
        pl.kernel,
        mesh=mesh,
        out_type=jax.ShapeDtypeStruct((t_len, d, b_len), jnp.float32),
        scratch_types=[
            pltpu.VMEM((128,), jnp.int32),
            pltpu.VMEM((128,), jnp.int32),
            pltpu.VMEM((128,), jnp.int32),
            pltpu.VMEM((128,), jnp.int32),
            pltpu.VMEM((128, 128), jnp.float32),
            pltpu.VMEM((128, 128), jnp.float32),
            pltpu.VMEM((d, 128), jnp.float32),
            pltpu.VMEM((d, 128), jnp.float32),
            pltpu.SemaphoreType.DMA,
            pltpu.SemaphoreType.DMA,
            pltpu.SemaphoreType.DMA,
            pltpu.SemaphoreType.DMA,
        ],
        compiler_params=pltpu.CompilerParams(needs_layout_passes=False),
    )
    def lookup_kernel(idx_hbm, t4_hbm, out_hbm,
                      i0, i1, q0, q1, g0, g1, o0, o1, sg0, sg1, so0, so1):
        wid = lax.axis_index("s") * nc + lax.axis_index("c")
        base = wid * per
        idxb, qb = [i0, i1], [q0, q1]
        gbuf, obuf = [g0, g1], [o0, o1]
        sg, so = [sg0, sg1], [so0, so1]

        def tj(n):
            blk = base + n
            return lax.div(blk, nb), lax.rem(blk, nb)

        def fetch(n, b):
            t, j = tj(n)
            pltpu.sync_copy(idx_hbm.at[t, pl.ds(j * 128, 128)], idxb[b])
            for h in range(8):
                vals = idxb[b][pl.ds(LANES * h, LANES)]
                qb[b][pl.ds(LANES * h, LANES)] = (
                    lax.shift_right_logical(vals, 2))
            pltpu.async_copy(t4_hbm.at[qb[b]], gbuf[b], sg[b])

        def wait_gather(b):
            pltpu.make_async_copy(t4_hbm.at[qb[b]], gbuf[b], sg[b]).wait()

        def extract(b):
            for h in range(8):
                vals = idxb[b][pl.ds(LANES * h, LANES)]
                rows = _iota16() + LANES * h
                colbase = (vals & 3) * 32
                for c in range(d):
                    obuf[b][c, pl.ds(LANES * h, LANES)] = (
                        plsc.load_gather(gbuf[b], [rows, colbase + c]))

        def start_out(n, b):
            t, j = tj(n)
            pltpu.async_copy(obuf[b], out_hbm.at[t, :, pl.ds(j * 128, 128)],
                             so[b])

        def wait_out(b):
            pltpu.make_async_copy(obuf[b], out_hbm.at[0, :, pl.ds(0, 128)],
                                  so[b]).wait()

        fetch(0, 0)

        @pl.loop(0, per)
        def _blocks(n):
            b = lax.rem(n, 2)
            for bb in range(2):
                @pl.when(b == bb)
                def _():
                    @pl.when(n + 1 < per)
                    def _():
                        fetch(n + 1, 1 - bb)

                    wait_gather(bb)

                    @pl.when(n >= 2)
                    def _():
                        wait_out(bb)

                    extract(bb)
                    start_out(n, bb)

        wait_out(0)
        wait_out(1)

    return lookup_kernel


@jax.jit
def kernel(input, weight):
    b_len, t_len = input.shape
    v, d = weight.shape
    w2 = weight.T
    idx2 = input.T.astype(jnp.int32)
    t4 = _make_pack(v, d)(w2)
    out_t = _make_lookup(t_len, b_len, d)(idx2, t4)
    return jnp.transpose(out_t, (2, 0, 1))
